# initial kernel scaffold (unmeasured)
import jax
import jax.numpy as jnp
from jax import lax
from jax.experimental import pallas as pl
from jax.experimental.pallas import tpu as pltpu


def kernel(
    x,
):
    def body(*refs):
        pass

    out_shape = jax.ShapeDtypeStruct(..., jnp.float32)
    return pl.pallas_call(body, out_shape=out_shape)(...)


# baseline (device time: 44577 ns/iter reference)
import jax
import jax.numpy as jnp
from jax import lax
from jax.experimental import pallas as pl
from jax.experimental.pallas import tpu as pltpu

N_DEV = 4


def kernel(x):
    m, n = x.shape

    def body(x_ref, out_ref, comm_ref, send_sems, recv_sems):
        my_pos = lax.axis_index("i")
        left = (my_pos - 1) % N_DEV
        right = (my_pos + 1) % N_DEV

        barrier_sem = pltpu.get_barrier_semaphore()
        for nbr in (left, right):
            pl.semaphore_signal(
                barrier_sem, inc=1,
                device_id=(nbr,), device_id_type=pl.DeviceIdType.MESH,
            )
        pl.semaphore_wait(barrier_sem, 2)

        comm_ref[0, :, :] = x_ref[:, :]
        out_ref[:, :] = x_ref[:, :]

        for h in range(N_DEV - 1):
            rdma = pltpu.make_async_remote_copy(
                src_ref=comm_ref.at[h],
                dst_ref=comm_ref.at[h + 1],
                send_sem=send_sems.at[h],
                recv_sem=recv_sems.at[h],
                device_id=(right,),
                device_id_type=pl.DeviceIdType.MESH,
            )
            rdma.start()
            rdma.wait()
            out_ref[:, :] += comm_ref[h + 1, :, :]

    return pl.pallas_call(
        body,
        out_shape=jax.ShapeDtypeStruct((m, n), x.dtype),
        in_specs=[pl.BlockSpec(memory_space=pltpu.VMEM)],
        out_specs=pl.BlockSpec(memory_space=pltpu.VMEM),
        scratch_shapes=[
            pltpu.VMEM((N_DEV, m, n), x.dtype),
            pltpu.SemaphoreType.DMA((N_DEV - 1,)),
            pltpu.SemaphoreType.DMA((N_DEV - 1,)),
        ],
        compiler_params=pltpu.CompilerParams(collective_id=0),
    )(x)


# device time: 18957 ns/iter; 2.3515x vs baseline; 2.3515x over previous
import jax
import jax.numpy as jnp
from jax import lax
from jax.experimental import pallas as pl
from jax.experimental.pallas import tpu as pltpu

N_DEV = 4


def kernel(x):
    m, n = x.shape
    q = m // 4

    def body(x_ref, out_ref, comm_ref, send_sems, recv_sems):
        p = lax.axis_index("i")
        py = jnp.bitwise_xor(p, 1)
        px = jnp.bitwise_xor(p, 3)

        barrier_sem = pltpu.get_barrier_semaphore()
        for nbr in (py, px):
            pl.semaphore_signal(
                barrier_sem, inc=1,
                device_id=(nbr,), device_id_type=pl.DeviceIdType.MESH,
            )
        pl.semaphore_wait(barrier_sem, 2)

        gray = jnp.bitwise_and(jnp.bitwise_xor(p, p // 2), 1)
        bit1 = jnp.bitwise_and(p // 2, 1)

        plans = [
            (py, px, 0, gray),
            (px, py, 2 * q, bit1),
        ]
        rdmas = {}

        for b, (p1, _p2, base, keep) in enumerate(plans):
            r = pltpu.make_async_remote_copy(
                src_ref=x_ref.at[pl.ds(base + (1 - keep) * q, q)],
                dst_ref=comm_ref.at[b, 0],
                send_sem=send_sems.at[b, 0],
                recv_sem=recv_sems.at[b, 0],
                device_id=(p1,),
                device_id_type=pl.DeviceIdType.MESH,
            )
            r.start()
            rdmas[b, 0] = r

        for b, (_p1, p2, base, keep) in enumerate(plans):
            keep_sl = pl.ds(base + keep * q, q)
            rdmas[b, 0].wait()
            out_ref[keep_sl, :] = x_ref[keep_sl, :] + comm_ref[b, 0, :, :]
            r = pltpu.make_async_remote_copy(
                src_ref=out_ref.at[keep_sl],
                dst_ref=comm_ref.at[b, 1],
                send_sem=send_sems.at[b, 1],
                recv_sem=recv_sems.at[b, 1],
                device_id=(p2,),
                device_id_type=pl.DeviceIdType.MESH,
            )
            r.start()
            rdmas[b, 1] = r

        for b, (p1, _p2, base, keep) in enumerate(plans):
            keep_sl = pl.ds(base + keep * q, q)
            rdmas[b, 1].wait()
            out_ref[keep_sl, :] += comm_ref[b, 1, :, :]
            r = pltpu.make_async_remote_copy(
                src_ref=out_ref.at[keep_sl],
                dst_ref=out_ref.at[keep_sl],
                send_sem=send_sems.at[b, 2],
                recv_sem=recv_sems.at[b, 2],
                device_id=(p1,),
                device_id_type=pl.DeviceIdType.MESH,
            )
            r.start()
            rdmas[b, 2] = r

        for b in range(2):
            rdmas[b, 2].wait()

    return pl.pallas_call(
        body,
        out_shape=jax.ShapeDtypeStruct((m, n), x.dtype),
        in_specs=[pl.BlockSpec(memory_space=pltpu.VMEM)],
        out_specs=pl.BlockSpec(memory_space=pltpu.VMEM),
        scratch_shapes=[
            pltpu.VMEM((2, 2, q, n), x.dtype),
            pltpu.SemaphoreType.DMA((2, 3)),
            pltpu.SemaphoreType.DMA((2, 3)),
        ],
        compiler_params=pltpu.CompilerParams(collective_id=0),
    )(x)


# device time: 16224 ns/iter; 2.7476x vs baseline; 1.1685x over previous
import jax
import jax.numpy as jnp
from jax import lax
from jax.experimental import pallas as pl
from jax.experimental.pallas import tpu as pltpu

N_DEV = 4
NCHUNK = 2


def kernel(x):
    m, n = x.shape
    q = m // 4
    qc = q // NCHUNK

    def body(x_ref, out_ref, comm_ref, send_sems, recv_sems):
        p = lax.axis_index("i")
        py = jnp.bitwise_xor(p, 1)
        px = jnp.bitwise_xor(p, 3)

        barrier_sem = pltpu.get_barrier_semaphore()
        for nbr in (py, px):
            pl.semaphore_signal(
                barrier_sem, inc=1,
                device_id=(nbr,), device_id_type=pl.DeviceIdType.MESH,
            )
        pl.semaphore_wait(barrier_sem, 2)

        gray = jnp.bitwise_and(jnp.bitwise_xor(p, p // 2), 1)
        bit1 = jnp.bitwise_and(p // 2, 1)

        plans = [
            (py, px, 0, gray),
            (px, py, 2 * q, bit1),
        ]
        rdmas = {}

        for b, (p1, _p2, base, keep) in enumerate(plans):
            for c in range(NCHUNK):
                r = pltpu.make_async_remote_copy(
                    src_ref=x_ref.at[pl.ds(base + (1 - keep) * q + c * qc, qc)],
                    dst_ref=comm_ref.at[b, 0, c],
                    send_sem=send_sems.at[b, 0, c],
                    recv_sem=recv_sems.at[b, 0, c],
                    device_id=(p1,),
                    device_id_type=pl.DeviceIdType.MESH,
                )
                r.start()
                rdmas[b, 0, c] = r

        for c in range(NCHUNK):
            for b, (_p1, p2, base, keep) in enumerate(plans):
                keep_sl = pl.ds(base + keep * q + c * qc, qc)
                rdmas[b, 0, c].wait()
                out_ref[keep_sl, :] = x_ref[keep_sl, :] + comm_ref[b, 0, c, :, :]
                r = pltpu.make_async_remote_copy(
                    src_ref=out_ref.at[keep_sl],
                    dst_ref=comm_ref.at[b, 1, c],
                    send_sem=send_sems.at[b, 1, c],
                    recv_sem=recv_sems.at[b, 1, c],
                    device_id=(p2,),
                    device_id_type=pl.DeviceIdType.MESH,
                )
                r.start()
                rdmas[b, 1, c] = r

        for c in range(NCHUNK):
            for b, (p1, _p2, base, keep) in enumerate(plans):
                keep_sl = pl.ds(base + keep * q + c * qc, qc)
                rdmas[b, 1, c].wait()
                out_ref[keep_sl, :] += comm_ref[b, 1, c, :, :]
                r = pltpu.make_async_remote_copy(
                    src_ref=out_ref.at[keep_sl],
                    dst_ref=out_ref.at[keep_sl],
                    send_sem=send_sems.at[b, 2, c],
                    recv_sem=recv_sems.at[b, 2, c],
                    device_id=(p1,),
                    device_id_type=pl.DeviceIdType.MESH,
                )
                r.start()
                rdmas[b, 2, c] = r

        for c in range(NCHUNK):
            for b in range(2):
                rdmas[b, 2, c].wait()

    return pl.pallas_call(
        body,
        out_shape=jax.ShapeDtypeStruct((m, n), x.dtype),
        in_specs=[pl.BlockSpec(memory_space=pltpu.VMEM)],
        out_specs=pl.BlockSpec(memory_space=pltpu.VMEM),
        scratch_shapes=[
            pltpu.VMEM((2, 2, NCHUNK, qc, n), x.dtype),
            pltpu.SemaphoreType.DMA((2, 3, NCHUNK)),
            pltpu.SemaphoreType.DMA((2, 3, NCHUNK)),
        ],
        compiler_params=pltpu.CompilerParams(collective_id=0),
    )(x)


# device time: 15941 ns/iter; 2.7964x vs baseline; 1.0178x over previous
import jax
import jax.numpy as jnp
from jax import lax
from jax.experimental import pallas as pl
from jax.experimental.pallas import tpu as pltpu

N_DEV = 4
NCHUNK = 4


def kernel(x):
    m, n = x.shape
    q = m // 4
    qc = q // NCHUNK

    def body(x_ref, out_ref, comm_ref, w_ref, send_sems, recv_sems):
        p = lax.axis_index("i")
        py = jnp.bitwise_xor(p, 1)
        px = jnp.bitwise_xor(p, 3)

        barrier_sem = pltpu.get_barrier_semaphore()
        for nbr in (py, px):
            pl.semaphore_signal(
                barrier_sem, inc=1,
                device_id=(nbr,), device_id_type=pl.DeviceIdType.MESH,
            )
        pl.semaphore_wait(barrier_sem, 2)

        gray = jnp.bitwise_and(jnp.bitwise_xor(p, p // 2), 1)
        bit1 = jnp.bitwise_and(p // 2, 1)

        plans = [
            (py, px, 0, gray),
            (px, py, 2 * q, bit1),
        ]
        rdmas = {}

        for b, (p1, _p2, base, keep) in enumerate(plans):
            for c in range(NCHUNK):
                r = pltpu.make_async_remote_copy(
                    src_ref=x_ref.at[pl.ds(base + (1 - keep) * q + c * qc, qc)],
                    dst_ref=comm_ref.at[b, 0, c],
                    send_sem=send_sems.at[b, 0, c],
                    recv_sem=recv_sems.at[b, 0, c],
                    device_id=(p1,),
                    device_id_type=pl.DeviceIdType.MESH,
                )
                r.start()
                rdmas[b, 0, c] = r

        for c in range(NCHUNK):
            for b, (_p1, p2, base, keep) in enumerate(plans):
                keep_sl = pl.ds(base + keep * q + c * qc, qc)
                rdmas[b, 0, c].wait_recv()
                w_ref[b, c, :, :] = x_ref[keep_sl, :] + comm_ref[b, 0, c, :, :]
                r = pltpu.make_async_remote_copy(
                    src_ref=w_ref.at[b, c],
                    dst_ref=comm_ref.at[b, 1, c],
                    send_sem=send_sems.at[b, 1, c],
                    recv_sem=recv_sems.at[b, 1, c],
                    device_id=(p2,),
                    device_id_type=pl.DeviceIdType.MESH,
                )
                r.start()
                rdmas[b, 1, c] = r

        for c in range(NCHUNK):
            for b, (p1, _p2, base, keep) in enumerate(plans):
                keep_sl = pl.ds(base + keep * q + c * qc, qc)
                rdmas[b, 1, c].wait_recv()
                out_ref[keep_sl, :] = w_ref[b, c, :, :] + comm_ref[b, 1, c, :, :]
                r = pltpu.make_async_remote_copy(
                    src_ref=out_ref.at[keep_sl],
                    dst_ref=out_ref.at[keep_sl],
                    send_sem=send_sems.at[b, 2, c],
                    recv_sem=recv_sems.at[b, 2, c],
                    device_id=(p1,),
                    device_id_type=pl.DeviceIdType.MESH,
                )
                r.start()
                rdmas[b, 2, c] = r

        for c in range(NCHUNK):
            for b in range(2):
                rdmas[b, 2, c].wait_recv()
        for key, r in rdmas.items():
            r.wait_send()

    return pl.pallas_call(
        body,
        out_shape=jax.ShapeDtypeStruct((m, n), x.dtype),
        in_specs=[pl.BlockSpec(memory_space=pltpu.VMEM)],
        out_specs=pl.BlockSpec(memory_space=pltpu.VMEM),
        scratch_shapes=[
            pltpu.VMEM((2, 2, NCHUNK, qc, n), x.dtype),
            pltpu.VMEM((2, NCHUNK, qc, n), x.dtype),
            pltpu.SemaphoreType.DMA((2, 3, NCHUNK)),
            pltpu.SemaphoreType.DMA((2, 3, NCHUNK)),
        ],
        compiler_params=pltpu.CompilerParams(collective_id=0),
    )(x)


# device time: 15893 ns/iter; 2.8048x vs baseline; 1.0030x over previous
import jax
import jax.numpy as jnp
from jax import lax
from jax.experimental import pallas as pl
from jax.experimental.pallas import tpu as pltpu

N_DEV = 4
NCHUNK = 4


def kernel(x):
    m, n = x.shape
    q = m // 4
    qc = q // NCHUNK

    def body(x_ref, out_ref, comm_ref, w_ref, send_sems, recv_sems):
        p = lax.axis_index("i")
        py = jnp.bitwise_xor(p, 1)
        px = jnp.bitwise_xor(p, 3)

        barrier_sem = pltpu.get_barrier_semaphore()
        for nbr in (py, px):
            pl.semaphore_signal(
                barrier_sem, inc=1,
                device_id=(nbr,), device_id_type=pl.DeviceIdType.MESH,
            )
        pl.semaphore_wait(barrier_sem, 2)

        gray = jnp.bitwise_and(jnp.bitwise_xor(p, p // 2), 1)
        bit1 = jnp.bitwise_and(p // 2, 1)

        plans = [
            (py, px, 0, gray),
            (px, py, 2 * q, bit1),
        ]

        def branch_on(keep, fn):
            for kv in (0, 1):
                pl.when(keep == kv)(lambda kv=kv: fn(kv))

        rdmas = {}

        for b, (p1, _p2, base, keep) in enumerate(plans):
            for c in range(NCHUNK):

                def start_s1(kv, b=b, p1=p1, base=base, c=c):
                    src_row = base + (1 - kv) * q + c * qc
                    pltpu.make_async_remote_copy(
                        src_ref=x_ref.at[pl.ds(src_row, qc)],
                        dst_ref=comm_ref.at[b, 0, c],
                        send_sem=send_sems.at[b, 0, c],
                        recv_sem=recv_sems.at[b, 0, c],
                        device_id=(p1,),
                        device_id_type=pl.DeviceIdType.MESH,
                    ).start()

                branch_on(keep, start_s1)
                rdmas[b, 0, c] = pltpu.make_async_remote_copy(
                    src_ref=x_ref.at[pl.ds(base, qc)],
                    dst_ref=comm_ref.at[b, 0, c],
                    send_sem=send_sems.at[b, 0, c],
                    recv_sem=recv_sems.at[b, 0, c],
                    device_id=(p1,),
                    device_id_type=pl.DeviceIdType.MESH,
                )

        for c in range(NCHUNK):
            for b, (p1, p2, base, keep) in enumerate(plans):
                rdmas[b, 0, c].wait_recv()

                def reduce_s1(kv, b=b, base=base, c=c):
                    row = base + kv * q + c * qc
                    w_ref[b, c, :, :] = (
                        x_ref[row:row + qc, :] + comm_ref[b, 0, c, :, :]
                    )

                branch_on(keep, reduce_s1)
                r = pltpu.make_async_remote_copy(
                    src_ref=w_ref.at[b, c],
                    dst_ref=comm_ref.at[b, 1, c],
                    send_sem=send_sems.at[b, 1, c],
                    recv_sem=recv_sems.at[b, 1, c],
                    device_id=(p2,),
                    device_id_type=pl.DeviceIdType.MESH,
                )
                r.start()
                rdmas[b, 1, c] = r

        for c in range(NCHUNK):
            for b, (p1, _p2, base, keep) in enumerate(plans):
                rdmas[b, 1, c].wait_recv()

                def reduce_and_s3(kv, b=b, p1=p1, base=base, c=c):
                    row = base + kv * q + c * qc
                    out_ref[row:row + qc, :] = (
                        w_ref[b, c, :, :] + comm_ref[b, 1, c, :, :]
                    )
                    pltpu.make_async_remote_copy(
                        src_ref=out_ref.at[pl.ds(row, qc)],
                        dst_ref=out_ref.at[pl.ds(row, qc)],
                        send_sem=send_sems.at[b, 2, c],
                        recv_sem=recv_sems.at[b, 2, c],
                        device_id=(p1,),
                        device_id_type=pl.DeviceIdType.MESH,
                    ).start()

                branch_on(keep, reduce_and_s3)
                rdmas[b, 2, c] = pltpu.make_async_remote_copy(
                    src_ref=out_ref.at[pl.ds(base, qc)],
                    dst_ref=out_ref.at[pl.ds(base, qc)],
                    send_sem=send_sems.at[b, 2, c],
                    recv_sem=recv_sems.at[b, 2, c],
                    device_id=(p1,),
                    device_id_type=pl.DeviceIdType.MESH,
                )

        for c in range(NCHUNK):
            for b in range(2):
                rdmas[b, 2, c].wait_recv()
        for r in rdmas.values():
            r.wait_send()

    return pl.pallas_call(
        body,
        out_shape=jax.ShapeDtypeStruct((m, n), x.dtype),
        in_specs=[pl.BlockSpec(memory_space=pltpu.VMEM)],
        out_specs=pl.BlockSpec(memory_space=pltpu.VMEM),
        scratch_shapes=[
            pltpu.VMEM((2, 2, NCHUNK, qc, n), x.dtype),
            pltpu.VMEM((2, NCHUNK, qc, n), x.dtype),
            pltpu.SemaphoreType.DMA((2, 3, NCHUNK)),
            pltpu.SemaphoreType.DMA((2, 3, NCHUNK)),
        ],
        compiler_params=pltpu.CompilerParams(collective_id=0),
    )(x)
